# Initial kernel scaffold; baseline (speedup 1.0000x reference)
import jax
import jax.numpy as jnp
from jax.experimental import pallas as pl


def kernel(edge_index, edge_attr, params):
    raise NotImplementedError("write your pallas kernel here")


# speedup vs baseline: 2.9074x; 2.9074x over previous
import jax
import jax.numpy as jnp
from jax import lax
from jax.experimental import pallas as pl
from jax.experimental.pallas import tpu as pltpu
from jax.experimental.pallas import tpu_sc as plsc

NN = 50000
NE = 800000
H = 64
NP = 50048
CH = 128
EP = 802816
NCH = EP // CH
NC, NS = 2, 16
NW = NC * NS
CPW = NCH // NW
RPT = NP // NS
NB = NP // 16
EB = 2048
EG = EP // EB
EPS = 1e-5

_pcall = pl.pallas_call


def _sc_mesh():
    return plsc.VectorSubcoreMesh(core_axis_name="c", subcore_axis_name="s",
                                  num_cores=NC, num_subcores=NS)


def _sc_params():
    return pltpu.CompilerParams(use_tc_tiling_on_sc=False)


def _sc_deg_body(dstm, ones_h, zeros_h, out, idxd, onesv, acc):
    cid = lax.axis_index("c")
    sid = lax.axis_index("s")
    w = sid * NC + cid
    pltpu.sync_copy(zeros_h.at[pl.ds(sid * RPT, RPT)],
                    acc.at[pl.ds(sid * RPT, RPT)])
    pltpu.sync_copy(ones_h, onesv)
    pltpu.sync_copy(dstm.at[w], idxd)
    plsc.subcore_barrier()

    def step(j, carry):
        pltpu.sync_copy(onesv, acc.at[idxd.at[j]], add=True)
        return carry

    lax.fori_loop(0, CPW, step, 0)
    plsc.subcore_barrier()
    pltpu.sync_copy(acc.at[pl.ds(sid * RPT, RPT)],
                    out.at[cid].at[pl.ds(sid * RPT, RPT)])


def _run_sc_deg(dstm, ones16, zeros16):
    return pl.kernel(
        _sc_deg_body,
        out_type=jax.ShapeDtypeStruct((NC, NP, 16), jnp.float32),
        mesh=_sc_mesh(),
        compiler_params=_sc_params(),
        scratch_types=[
            pltpu.VMEM((CPW, CH), jnp.int32),
            pltpu.VMEM((CH, 16), jnp.float32),
            pltpu.VMEM_SHARED((NP, 16), jnp.float32),
        ],
    )(dstm, ones16, zeros16)


def _sc_gs16_body(srcm, dstm, tab_h, zeros_h, out, idxs, idxd, rows, acc):
    cid = lax.axis_index("c")
    sid = lax.axis_index("s")
    w = sid * NC + cid
    pltpu.sync_copy(zeros_h.at[pl.ds(sid * RPT, RPT)],
                    acc.at[pl.ds(sid * RPT, RPT)])
    pltpu.sync_copy(srcm.at[w], idxs)
    pltpu.sync_copy(dstm.at[w], idxd)
    plsc.subcore_barrier()

    def step(j, carry):
        pltpu.sync_copy(tab_h.at[idxs.at[j]], rows)
        pltpu.sync_copy(rows, acc.at[idxd.at[j]], add=True)
        return carry

    lax.fori_loop(0, CPW, step, 0)
    plsc.subcore_barrier()
    pltpu.sync_copy(acc.at[pl.ds(sid * RPT, RPT)],
                    out.at[cid].at[pl.ds(sid * RPT, RPT)])


def _run_sc_gs16(srcm, dstm, tab16, zeros16):
    return pl.kernel(
        _sc_gs16_body,
        out_type=jax.ShapeDtypeStruct((NC, NP, 16), jnp.float32),
        mesh=_sc_mesh(),
        compiler_params=_sc_params(),
        scratch_types=[
            pltpu.VMEM((CPW, CH), jnp.int32),
            pltpu.VMEM((CPW, CH), jnp.int32),
            pltpu.VMEM((CH, 16), jnp.float32),
            pltpu.VMEM_SHARED((NP, 16), jnp.float32),
        ],
    )(srcm, dstm, tab16, zeros16)


def _sc_gather32_body(idxm, tab_h, out, idxv, rows):
    cid = lax.axis_index("c")
    sid = lax.axis_index("s")
    w = sid * NC + cid
    base = w * CPW
    pltpu.sync_copy(idxm.at[w], idxv)

    def step(j, carry):
        pltpu.sync_copy(tab_h.at[idxv.at[j]], rows)
        pltpu.sync_copy(rows, out.at[pl.ds((base + j) * CH, CH)])
        return carry

    lax.fori_loop(0, CPW, step, 0)


def _run_sc_gather32(idxm, tab32):
    return pl.kernel(
        _sc_gather32_body,
        out_type=jax.ShapeDtypeStruct((EP, 32), jnp.float32),
        mesh=_sc_mesh(),
        compiler_params=_sc_params(),
        scratch_types=[
            pltpu.VMEM((CPW, CH), jnp.int32),
            pltpu.VMEM((CH, 32), jnp.float32),
        ],
    )(idxm, tab32)


def _full(shape):
    return pl.BlockSpec(shape, lambda i: (0, 0))


def _t0_body(ew4, e0w, a0, b0m, n0w, a1, b1m, n1w, e1w, nb0, embb, e0b, cb0,
             nb1, e1b, cb1, x0r, w0,
             owa0, owb0, owa1, owb1, on1, om0, ok0, ok1, oc):
    owa0[...] = jnp.dot(n0w[...][:H, :], a0[...])
    owb0[...] = jnp.dot(n0w[...][H:, :], a0[...])
    owa1[...] = jnp.dot(n1w[...][:H, :], a1[...])
    owb1[...] = jnp.dot(n1w[...][H:, :], a1[...])
    on1[...] = jnp.dot(e1w[...], b1m[...])
    om0[...] = jnp.dot(jnp.dot(ew4[...], e0w[...]), b0m[...])
    ok0[...] = (jnp.dot(nb0[...], a0[...])
                + jnp.dot(jnp.dot(embb[...], e0w[...]) + e0b[...], b0m[...])
                + cb0[...])
    ok1[...] = (jnp.dot(nb1[...], a1[...]) + jnp.dot(e1b[...], b1m[...])
                + cb1[...])
    oc[...] = jnp.dot(x0r[...], w0[...])


def _run_t0(p, ew4, x0r):
    return _pcall(
        _t0_body,
        grid=(1,),
        in_specs=[_full((4, H)), _full((H, H)), _full((H, H)), _full((H, H)),
                  _full((2 * H, H)), _full((H, H)), _full((H, H)),
                  _full((2 * H, H)), _full((H, H)),
                  _full((1, H)), _full((1, H)), _full((1, H)), _full((1, H)),
                  _full((1, H)), _full((1, H)), _full((1, H)),
                  _full((1, H)), _full((H, H))],
        out_specs=[_full((H, H)), _full((H, H)), _full((H, H)), _full((H, H)),
                   _full((H, H)), _full((4, H)), _full((1, H)), _full((1, H)),
                   _full((1, H))],
        out_shape=[jax.ShapeDtypeStruct((H, H), jnp.float32)] * 5
        + [jax.ShapeDtypeStruct((4, H), jnp.float32)]
        + [jax.ShapeDtypeStruct((1, H), jnp.float32)] * 3,
    )(ew4, p['eb0_edge_w'], p['eb0_comb_w'][:H], p['eb0_comb_w'][H:],
      p['eb0_node_w'], p['eb1_comb_w'][:H], p['eb1_comb_w'][H:],
      p['eb1_node_w'], p['eb1_edge_w'],
      p['eb0_node_b'].reshape(1, H), p['edge_emb_b'].reshape(1, H),
      p['eb0_edge_b'].reshape(1, H), p['eb0_comb_b'].reshape(1, H),
      p['eb1_node_b'].reshape(1, H), p['eb1_edge_b'].reshape(1, H),
      p['eb1_comb_b'].reshape(1, H),
      x0r, p['gcn0_w'])


def _t1_body(p0, p1, o):
    o[...] = lax.rsqrt(p0[...] + p1[...] + 1.0)


def _run_t1(deg_parts):
    rb = NP * 16 // 128
    p0 = deg_parts[0].reshape(rb, 128)
    p1 = deg_parts[1].reshape(rb, 128)
    o = _pcall(
        _t1_body,
        grid=(1,),
        in_specs=[_full((rb, 128))] * 2,
        out_specs=_full((rb, 128)),
        out_shape=jax.ShapeDtypeStruct((rb, 128), jnp.float32),
    )(p0, p1)
    return o.reshape(NP, 16)


def _t2a_body(dv, n0, n1, o, accv):
    i = pl.program_id(0)

    @pl.when(i == 0)
    def _():
        accv[...] = jnp.zeros_like(accv)

    d = dv[...][:, :1]
    nn = n0[...][:, :1] + n1[...][:, :1]
    a = d * (nn + d)
    rid = lax.broadcasted_iota(jnp.int32, (NB, 1), 0) + i * NB
    am = jnp.where(rid < NN, a, 0.0)
    ps = jnp.sum(am)
    psq = jnp.sum(am * am)
    lane = lax.broadcasted_iota(jnp.int32, (1, 128), 1)
    accv[...] += (jnp.where(lane == 0, ps, 0.0)
                  + jnp.where(lane == 1, psq, 0.0))

    @pl.when(i == pl.num_programs(0) - 1)
    def _():
        o[...] = accv[...]


def _run_t2a(dinv16, nparts):
    spec16 = pl.BlockSpec((NB, 16), lambda i: (i, 0))
    return _pcall(
        _t2a_body,
        grid=(16,),
        in_specs=[spec16, spec16, spec16],
        out_specs=pl.BlockSpec((1, 128), lambda i: (0, 0)),
        out_shape=jax.ShapeDtypeStruct((1, 128), jnp.float32),
        scratch_shapes=[pltpu.VMEM((1, 128), jnp.float32)],
    )(dinv16, nparts[0], nparts[1])


def _t2b_body(dv, n0, n1, st, cv, g0, b0, w1, wa0, wb0,
              oy0, oy1, oy2, oy3, oxw, op0a, op0b, oq0a, oq0b):
    d = dv[...][:, :1]
    nn = n0[...][:, :1] + n1[...][:, :1]
    a = d * (nn + d)
    amv = st[...][:, 0:1] / NN
    avv = st[...][:, 1:2] / NN - amv * amv
    c = cv[...]
    u = (c * g0[...]) * lax.rsqrt(avv * c * c + EPS)
    x1 = jnp.maximum((a - amv) * u + b0[...], 0.0)
    xw = jnp.dot(x1, w1[...])
    y1 = xw * d
    oy0[...] = y1[:, 0:16]
    oy1[...] = y1[:, 16:32]
    oy2[...] = y1[:, 32:48]
    oy3[...] = y1[:, 48:64]
    oxw[...] = xw
    p0 = jnp.dot(x1, wa0[...])
    q0 = jnp.dot(x1, wb0[...])
    op0a[...] = p0[:, :32]
    op0b[...] = p0[:, 32:]
    oq0a[...] = q0[:, :32]
    oq0b[...] = q0[:, 32:]


def _run_t2b(dinv16, nparts, st, cvec, g0, b0, w1, wa0, wb0):
    spec16 = pl.BlockSpec((NB, 16), lambda i: (i, 0))
    spec32 = pl.BlockSpec((NB, 32), lambda i: (i, 0))
    spec64 = pl.BlockSpec((NB, H), lambda i: (i, 0))
    sfull = [pl.BlockSpec(s, lambda i: (0, 0))
             for s in [(1, 128), (1, H), (1, H), (1, H), (H, H), (H, H),
                       (H, H)]]
    return _pcall(
        _t2b_body,
        grid=(16,),
        in_specs=[spec16, spec16, spec16] + sfull,
        out_specs=[spec16] * 4 + [spec64] + [spec32] * 4,
        out_shape=[jax.ShapeDtypeStruct((NP, 16), jnp.float32)] * 4
        + [jax.ShapeDtypeStruct((NP, H), jnp.float32)]
        + [jax.ShapeDtypeStruct((NP, 32), jnp.float32)] * 4,
    )(dinv16, nparts[0], nparts[1], st, cvec,
      g0.reshape(1, H), b0.reshape(1, H), w1, wa0, wb0)


def _t3a_body(q0a, q0b, q1a, q1b, q2a, q2b, q3a, q3b, xw, dv, b1,
              og, osum, osq, accs, accq):
    i = pl.program_id(0)

    @pl.when(i == 0)
    def _():
        accs[...] = jnp.zeros_like(accs)
        accq[...] = jnp.zeros_like(accq)

    d = dv[...][:, :1]
    sac = jnp.concatenate([q0a[...] + q0b[...], q1a[...] + q1b[...],
                           q2a[...] + q2b[...], q3a[...] + q3b[...]], axis=1)
    g = d * sac + (d * d) * xw[...] + b1[...]
    og[...] = g
    rid = lax.broadcasted_iota(jnp.int32, (NB, 1), 0) + i * NB
    gm = jnp.where(rid < NN, g, 0.0)
    accs[...] += jnp.sum(gm, axis=0, keepdims=True)
    accq[...] += jnp.sum(gm * gm, axis=0, keepdims=True)

    @pl.when(i == pl.num_programs(0) - 1)
    def _():
        osum[...] = accs[...]
        osq[...] = accq[...]


def _run_t3a(saccs, xw1, dinv16, b1):
    spec16 = pl.BlockSpec((NB, 16), lambda i: (i, 0))
    spec64 = pl.BlockSpec((NB, H), lambda i: (i, 0))
    sview = pl.BlockSpec((1, H), lambda i: (0, 0))
    args = []
    for sq in saccs:
        args += [sq[0], sq[1]]
    return _pcall(
        _t3a_body,
        grid=(16,),
        in_specs=[spec16] * 8 + [spec64, spec16, sview],
        out_specs=[spec64, sview, sview],
        out_shape=[jax.ShapeDtypeStruct((NP, H), jnp.float32),
                   jax.ShapeDtypeStruct((1, H), jnp.float32),
                   jax.ShapeDtypeStruct((1, H), jnp.float32)],
        scratch_shapes=[pltpu.VMEM((1, H), jnp.float32),
                        pltpu.VMEM((1, H), jnp.float32)],
    )(*args, xw1, dinv16, b1.reshape(1, H))


def _t3b_body(g, ssum, ssq, gg, gb, wa1, wb1,
              op1a, op1b, oq1a, oq1b, opool, accp):
    i = pl.program_id(0)

    @pl.when(i == 0)
    def _():
        accp[...] = jnp.zeros_like(accp)

    mu = ssum[...] / NN
    var = ssq[...] / NN - mu * mu
    s = gg[...] * lax.rsqrt(var + EPS)
    t = gb[...] - mu * s
    x2 = jnp.maximum(g[...] * s + t, 0.0)
    p1 = jnp.dot(x2, wa1[...])
    q1 = jnp.dot(x2, wb1[...])
    op1a[...] = p1[:, :32]
    op1b[...] = p1[:, 32:]
    oq1a[...] = q1[:, :32]
    oq1b[...] = q1[:, 32:]
    rid = lax.broadcasted_iota(jnp.int32, (NB, 1), 0) + i * NB
    accp[...] += jnp.sum(jnp.where(rid < NN, x2, 0.0), axis=0, keepdims=True)

    @pl.when(i == pl.num_programs(0) - 1)
    def _():
        opool[...] = accp[...]


def _run_t3b(g1, ssum, ssq, gg, gb, wa1, wb1):
    spec32 = pl.BlockSpec((NB, 32), lambda i: (i, 0))
    spec64 = pl.BlockSpec((NB, H), lambda i: (i, 0))
    sview = pl.BlockSpec((1, H), lambda i: (0, 0))
    mview = pl.BlockSpec((H, H), lambda i: (0, 0))
    return _pcall(
        _t3b_body,
        grid=(16,),
        in_specs=[spec64, sview, sview, sview, sview, mview, mview],
        out_specs=[spec32] * 4 + [sview],
        out_shape=[jax.ShapeDtypeStruct((NP, 32), jnp.float32)] * 4
        + [jax.ShapeDtypeStruct((1, H), jnp.float32)],
        scratch_shapes=[pltpu.VMEM((1, H), jnp.float32)],
    )(g1, ssum, ssq, gg.reshape(1, H), gb.reshape(1, H), wa1, wb1)


def _e1_body(gpa, gpb, gqa, gqb, at4, m0, k0, oc, osum, osq, accs, accq):
    i = pl.program_id(0)

    @pl.when(i == 0)
    def _():
        accs[...] = jnp.zeros_like(accs)
        accq[...] = jnp.zeros_like(accq)

    gp = jnp.concatenate([gpa[...], gpb[...]], axis=1)
    gq = jnp.concatenate([gqa[...], gqb[...]], axis=1)
    cb = gp + gq + jnp.dot(at4[...], m0[...]) + k0[...]
    oc[...] = cb
    rid = lax.broadcasted_iota(jnp.int32, (EB, 1), 0) + i * EB
    cm = jnp.where(rid < NE, cb, 0.0)
    accs[...] += jnp.sum(cm, axis=0, keepdims=True)
    accq[...] += jnp.sum(cm * cm, axis=0, keepdims=True)

    @pl.when(i == pl.num_programs(0) - 1)
    def _():
        osum[...] = accs[...]
        osq[...] = accq[...]


def _run_e1(gp0, gq0, at4, m0, k0):
    espec = pl.BlockSpec((EB, H), lambda i: (i, 0))
    hspec = pl.BlockSpec((EB, 32), lambda i: (i, 0))
    aspec = pl.BlockSpec((EB, 4), lambda i: (i, 0))
    sview = pl.BlockSpec((1, H), lambda i: (0, 0))
    m0view = pl.BlockSpec((4, H), lambda i: (0, 0))
    return _pcall(
        _e1_body,
        grid=(EG,),
        in_specs=[hspec, hspec, hspec, hspec, aspec, m0view, sview],
        out_specs=[espec, sview, sview],
        out_shape=[jax.ShapeDtypeStruct((EP, H), jnp.float32),
                   jax.ShapeDtypeStruct((1, H), jnp.float32),
                   jax.ShapeDtypeStruct((1, H), jnp.float32)],
        scratch_shapes=[pltpu.VMEM((1, H), jnp.float32),
                        pltpu.VMEM((1, H), jnp.float32)],
    )(gp0[0], gp0[1], gq0[0], gq0[1], at4, m0, k0)


def _e3_body(c0, s0sum, s0sq, g0e, b0e, gpa, gpb, gqa, gqb, n1m, k1,
             oc, osum, osq, accs, accq):
    i = pl.program_id(0)

    @pl.when(i == 0)
    def _():
        accs[...] = jnp.zeros_like(accs)
        accq[...] = jnp.zeros_like(accq)

    mu = s0sum[...] / NE
    var = s0sq[...] / NE - mu * mu
    s = g0e[...] * lax.rsqrt(var + EPS)
    t = b0e[...] - mu * s
    e1 = jnp.maximum(c0[...] * s + t, 0.0)
    gp = jnp.concatenate([gpa[...], gpb[...]], axis=1)
    gq = jnp.concatenate([gqa[...], gqb[...]], axis=1)
    cb = jnp.dot(e1, n1m[...]) + gp + gq + k1[...]
    oc[...] = cb
    rid = lax.broadcasted_iota(jnp.int32, (EB, 1), 0) + i * EB
    cm = jnp.where(rid < NE, cb, 0.0)
    accs[...] += jnp.sum(cm, axis=0, keepdims=True)
    accq[...] += jnp.sum(cm * cm, axis=0, keepdims=True)

    @pl.when(i == pl.num_programs(0) - 1)
    def _():
        osum[...] = accs[...]
        osq[...] = accq[...]


def _run_e3(comb0, s0sum, s0sq, g0e, b0e, gp1, gq1, n1m, k1):
    espec = pl.BlockSpec((EB, H), lambda i: (i, 0))
    hspec = pl.BlockSpec((EB, 32), lambda i: (i, 0))
    sview = pl.BlockSpec((1, H), lambda i: (0, 0))
    mview = pl.BlockSpec((H, H), lambda i: (0, 0))
    return _pcall(
        _e3_body,
        grid=(EG,),
        in_specs=[espec, sview, sview, sview, sview, hspec, hspec, hspec,
                  hspec, mview, sview],
        out_specs=[espec, sview, sview],
        out_shape=[jax.ShapeDtypeStruct((EP, H), jnp.float32),
                   jax.ShapeDtypeStruct((1, H), jnp.float32),
                   jax.ShapeDtypeStruct((1, H), jnp.float32)],
        scratch_shapes=[pltpu.VMEM((1, H), jnp.float32),
                        pltpu.VMEM((1, H), jnp.float32)],
    )(comb0, s0sum, s0sq, g0e.reshape(1, H), b0e.reshape(1, H),
      gp1[0], gp1[1], gq1[0], gq1[1], n1m, k1)


def _e4_body(c1, s1sum, s1sq, g1e, b1e, pol, polb, os_, omax, accm):
    i = pl.program_id(0)

    @pl.when(i == 0)
    def _():
        accm[...] = jnp.full_like(accm, -1e30)

    mu = s1sum[...] / NE
    var = s1sq[...] / NE - mu * mu
    s = g1e[...] * lax.rsqrt(var + EPS)
    t = b1e[...] - mu * s
    e2 = jnp.maximum(c1[...] * s + t, 0.0)
    sc = jnp.sum(e2 * pol[...], axis=1, keepdims=True) + polb[...]
    os_[...] = sc
    rid = lax.broadcasted_iota(jnp.int32, (EB, 1), 0) + i * EB
    scm = jnp.where(rid < NE, sc, -1e30)
    accm[...] = jnp.maximum(accm[...], jnp.max(scm, axis=0, keepdims=True))

    @pl.when(i == pl.num_programs(0) - 1)
    def _():
        omax[...] = accm[...]


def _run_e4(comb1, s1sum, s1sq, g1e, b1e, pol, polb):
    espec = pl.BlockSpec((EB, H), lambda i: (i, 0))
    cspec = pl.BlockSpec((EB, 1), lambda i: (i, 0))
    sview = pl.BlockSpec((1, H), lambda i: (0, 0))
    oview = pl.BlockSpec((1, 1), lambda i: (0, 0))
    return _pcall(
        _e4_body,
        grid=(EG,),
        in_specs=[espec, sview, sview, sview, sview, sview, oview],
        out_specs=[cspec, oview],
        out_shape=[jax.ShapeDtypeStruct((EP, 1), jnp.float32),
                   jax.ShapeDtypeStruct((1, 1), jnp.float32)],
        scratch_shapes=[pltpu.VMEM((1, 1), jnp.float32)],
    )(comb1, s1sum, s1sq, g1e.reshape(1, H), b1e.reshape(1, H),
      pol.reshape(1, H), polb.reshape(1, 1))


def _e5_body(sv, mx, oe, oz, accz):
    i = pl.program_id(0)

    @pl.when(i == 0)
    def _():
        accz[...] = jnp.zeros_like(accz)

    rid = lax.broadcasted_iota(jnp.int32, (EB, 1), 0) + i * EB
    ev = jnp.where(rid < NE, jnp.exp(sv[...] - mx[...]), 0.0)
    oe[...] = ev
    accz[...] += jnp.sum(ev, axis=0, keepdims=True)

    @pl.when(i == pl.num_programs(0) - 1)
    def _():
        oz[...] = accz[...]


def _run_e5(scores, mx):
    cspec = pl.BlockSpec((EB, 1), lambda i: (i, 0))
    oview = pl.BlockSpec((1, 1), lambda i: (0, 0))
    return _pcall(
        _e5_body,
        grid=(EG,),
        in_specs=[cspec, oview],
        out_specs=[cspec, oview],
        out_shape=[jax.ShapeDtypeStruct((EP, 1), jnp.float32),
                   jax.ShapeDtypeStruct((1, 1), jnp.float32)],
        scratch_shapes=[pltpu.VMEM((1, 1), jnp.float32)],
    )(scores, mx)


def _e6_body(ev, z, op):
    op[...] = ev[...] * (1.0 / z[...])


def _run_e6(expv, z):
    cspec = pl.BlockSpec((EB, 1), lambda i: (i, 0))
    oview = pl.BlockSpec((1, 1), lambda i: (0, 0))
    return _pcall(
        _e6_body,
        grid=(EG,),
        in_specs=[cspec, oview],
        out_specs=cspec,
        out_shape=jax.ShapeDtypeStruct((EP, 1), jnp.float32),
    )(expv, z)


def _t5_body(psum, w1, b1, w2r, b2, o):
    pooled = psum[...] / NN
    h = jnp.maximum(jnp.dot(pooled, w1[...]) + b1[...], 0.0)
    o[...] = jnp.tanh(jnp.sum(h * w2r[...], axis=1, keepdims=True) + b2[...])


def _run_t5(psum, w1, b1, w2, b2):
    return _pcall(
        _t5_body,
        grid=(1,),
        in_specs=[_full((1, H)), _full((H, 32)), _full((1, 32)),
                  _full((1, 32)), _full((1, 1))],
        out_specs=_full((1, 1)),
        out_shape=jax.ShapeDtypeStruct((1, 1), jnp.float32),
    )(psum, w1, b1.reshape(1, 32), w2.reshape(1, 32), b2.reshape(1, 1))


def kernel(edge_index, edge_attr, params):
    p = params
    src = edge_index[0].astype(jnp.int32)
    dst = edge_index[1].astype(jnp.int32)
    padn = EP - NE
    srcp = jnp.concatenate([src, jnp.zeros((padn,), jnp.int32)])
    dstp = jnp.concatenate([dst, jnp.full((padn,), NN, jnp.int32)])
    srcm32 = srcp.reshape(NW, CPW, CH)
    dstm32 = dstp.reshape(NW, CPW, CH)
    at4 = jnp.pad(edge_attr, ((0, padn), (0, 1)))
    ones16 = jnp.ones((CH, 16), jnp.float32)
    zeros16 = jnp.zeros((NP, 16), jnp.float32)
    ew4 = jnp.pad(p['edge_emb_w'], ((0, 1), (0, 0)))
    x0r = (p['node_emb_w'][0] + p['node_emb_b']).reshape(1, H)

    wa0, wb0, wa1, wb1, n1m, m0, k0, k1, cvec = _run_t0(p, ew4, x0r)

    deg_parts = _run_sc_deg(dstm32, ones16, zeros16)
    dinv16 = _run_t1(deg_parts)
    nparts = _run_sc_gs16(srcm32, dstm32, dinv16, zeros16)
    st = _run_t2a(dinv16, nparts)
    yq0, yq1, yq2, yq3, xw1, p0a, p0b, q0a, q0b = _run_t2b(
        dinv16, nparts, st, cvec, p['gcn0_gamma'], p['gcn0_beta'],
        p['gcn1_w'], wa0, wb0)

    saccs = [_run_sc_gs16(srcm32, dstm32, yq, zeros16)
             for yq in (yq0, yq1, yq2, yq3)]
    g1, gsum, gsq = _run_t3a(saccs, xw1, dinv16, p['gcn1_b'])
    p1a, p1b, q1a, q1b, psum = _run_t3b(
        g1, gsum, gsq, p['gcn1_gamma'], p['gcn1_beta'], wa1, wb1)

    gp0 = [_run_sc_gather32(srcm32, t) for t in (p0a, p0b)]
    gq0 = [_run_sc_gather32(dstm32, t) for t in (q0a, q0b)]
    comb0, s0sum, s0sq = _run_e1(gp0, gq0, at4, m0, k0)

    gp1 = [_run_sc_gather32(srcm32, t) for t in (p1a, p1b)]
    gq1 = [_run_sc_gather32(dstm32, t) for t in (q1a, q1b)]
    comb1, s1sum, s1sq = _run_e3(comb0, s0sum, s0sq, p['eb0_gamma'],
                                 p['eb0_beta'], gp1, gq1, n1m, k1)

    scores, mx = _run_e4(comb1, s1sum, s1sq, p['eb1_gamma'], p['eb1_beta'],
                         p['pol_w'], p['pol_b'])
    expv, z = _run_e5(scores, mx)
    pol2d = _run_e6(expv, z)
    policy = pol2d[:NE, 0]

    value = _run_t5(psum, p['val_w1'], p['val_b1'], p['val_w2'], p['val_b2'])
    return policy, value
